# baseline (speedup 1.0000x reference)
import functools

import jax
import jax.numpy as jnp
from jax import lax
from jax.experimental import pallas as pl
from jax.experimental.pallas import tpu as pltpu
from jax.experimental.pallas import tpu_sc as plsc

_LANES = 16
_CHUNK = 128
_NC = 2
_NS = 16
_NW = _NC * _NS


def _make_sc_kernel(B, E):
  b_per_w = B // _NW
  n_ch = b_per_w // _CHUNK
  n_col = E // _LANES
  mesh = plsc.VectorSubcoreMesh(core_axis_name="c", subcore_axis_name="s")

  def body(idx_u_hbm, idx_p_hbm, uemb_hbm, pemb_hbm, ubias_hbm, pbias_hbm,
           part_out, bsum_out,
           idx_u_v, idx_p_v, u_rows, p_rows, ub_v, pb_v, bsum_v, acc_v, sem):
    wid = lax.axis_index("s") * _NC + lax.axis_index("c")
    base = wid * b_per_w

    for j in range(n_ch):
      pltpu.sync_copy(idx_u_hbm.at[pl.ds(base + j * _CHUNK, _CHUNK)],
                      idx_u_v.at[j])
      pltpu.sync_copy(idx_p_hbm.at[pl.ds(base + j * _CHUNK, _CHUNK)],
                      idx_p_v.at[j])

    copies = []
    for j in range(n_ch):
      sl = pl.ds(j * _CHUNK, _CHUNK)
      copies.append(pltpu.async_copy(uemb_hbm.at[idx_u_v.at[j]],
                                     u_rows.at[sl], sem))
      copies.append(pltpu.async_copy(pemb_hbm.at[idx_p_v.at[j]],
                                     p_rows.at[sl], sem))
      copies.append(pltpu.async_copy(ubias_hbm.at[idx_u_v.at[j]],
                                     ub_v.at[sl], sem))
      copies.append(pltpu.async_copy(pbias_hbm.at[idx_p_v.at[j]],
                                     pb_v.at[sl], sem))
    for cp in copies:
      cp.wait()

    zero = jnp.zeros((_LANES,), jnp.float32)

    @plsc.parallel_loop(0, b_per_w, unroll=4, carry=(zero,) * n_col)
    def accs(r, acc):
      return tuple(
          acc[c] + u_rows[r, pl.ds(c * _LANES, _LANES)]
          * p_rows[r, pl.ds(c * _LANES, _LANES)]
          for c in range(n_col))

    total = zero
    for c in range(n_col):
      total = total + accs[c]
    acc_v[...] = total
    pltpu.sync_copy(acc_v, part_out.at[wid])

    @plsc.parallel_loop(0, b_per_w, step=_LANES)
    def _(i):
      sl = pl.ds(i, _LANES)
      bsum_v[sl] = ub_v[sl] + pb_v[sl]

    pltpu.sync_copy(bsum_v, bsum_out.at[pl.ds(base, b_per_w)])

  out_type = (
      jax.ShapeDtypeStruct((_NW, _LANES), jnp.float32),
      jax.ShapeDtypeStruct((B,), jnp.float32),
  )
  scratch = [
      pltpu.VMEM((n_ch, _CHUNK), jnp.int32),
      pltpu.VMEM((n_ch, _CHUNK), jnp.int32),
      pltpu.VMEM((b_per_w, E), jnp.float32),
      pltpu.VMEM((b_per_w, E), jnp.float32),
      pltpu.VMEM((b_per_w,), jnp.float32),
      pltpu.VMEM((b_per_w,), jnp.float32),
      pltpu.VMEM((b_per_w,), jnp.float32),
      pltpu.VMEM((_LANES,), jnp.float32),
      pltpu.SemaphoreType.DMA,
  ]
  return pl.kernel(body, out_type, mesh=mesh, scratch_types=scratch,
                   compiler_params=pltpu.CompilerParams(
                       use_tc_tiling_on_sc=False))


def _combine_body(part_ref, bias_ref, out_ref):
  total = jnp.sum(part_ref[...])
  out_ref[...] = jax.nn.sigmoid(bias_ref[...] + total)


def kernel(inputs, user_emb, user_bias, place_emb, place_bias):
  B = inputs.shape[0]
  E = user_emb.shape[1]
  idx_u = inputs[:, 0].astype(jnp.int32)
  idx_p = inputs[:, 1].astype(jnp.int32)
  ubias_flat = user_bias.reshape(-1)
  pbias_flat = place_bias.reshape(-1)

  parts, bias_sum = _make_sc_kernel(B, E)(
      idx_u, idx_p, user_emb, place_emb, ubias_flat, pbias_flat)

  rows = B // 128
  out2d = pl.pallas_call(
      _combine_body,
      out_shape=jax.ShapeDtypeStruct((rows, 128), jnp.float32),
  )(parts, bias_sum.reshape(rows, 128))
  return out2d.reshape(B, 1)
